# baseline (speedup 1.0000x reference)
import functools

import jax
import jax.numpy as jnp
from jax import lax
from jax.experimental import pallas as pl
from jax.experimental.pallas import tpu as pltpu
from jax.experimental.pallas import tpu_sc as plsc

NC = 2
NS = 16
L = 16
NW = NC * NS

BATCH = 16384
H_FEAT = 64
N_CLASSES = 2
BPW = BATCH // NW
CHUNK = 128
NCHUNK = BPW // CHUNK


def _edge_logits_body(src_hbm, dst_hbm, emb_hbm, w_hbm, b_hbm, out_hbm,
                      sidx, didx, srows, drows, wv, bv, outv, sem):
    wid = lax.axis_index("s") * NC + lax.axis_index("c")
    pltpu.sync_copy(src_hbm.at[pl.ds(wid * NCHUNK, NCHUNK)], sidx)
    pltpu.sync_copy(dst_hbm.at[pl.ds(wid * NCHUNK, NCHUNK)], didx)
    pltpu.sync_copy(w_hbm, wv)
    pltpu.sync_copy(b_hbm, bv)

    copies = []
    for c in range(NCHUNK):
        copies.append(pltpu.async_copy(
            emb_hbm.at[sidx.at[c]], srows.at[pl.ds(c * CHUNK, CHUNK)], sem))
        copies.append(pltpu.async_copy(
            emb_hbm.at[didx.at[c]], drows.at[pl.ds(c * CHUNK, CHUNK)], sem))
    for cp in copies:
        cp.wait()

    bvec = bv[...]
    b0 = bvec[0]
    b1 = bvec[1]
    w0c = [wv[0, pl.ds(k * L, L)] for k in range(H_FEAT // L)]
    w1c = [wv[1, pl.ds(k * L, L)] for k in range(H_FEAT // L)]
    lane = lax.iota(jnp.int32, L)

    def group_body(g, carry):
        rows = g * L + lane
        acc0 = jnp.full((L,), b0, jnp.float32)
        acc1 = jnp.full((L,), b1, jnp.float32)
        for j in range(H_FEAT):
            col = jnp.full((L,), j, jnp.int32)
            sv = plsc.load_gather(srows, [rows, col])
            dv = plsc.load_gather(drows, [rows, col])
            cd = sv * dv
            acc0 = acc0 + cd * w0c[j // L][j % L]
            acc1 = acc1 + cd * w1c[j // L][j % L]
        oidx = rows * N_CLASSES
        plsc.store_scatter(outv, [oidx], acc0)
        plsc.store_scatter(outv, [oidx + 1], acc1)
        return carry

    lax.fori_loop(0, BPW // L, group_body, 0)
    pltpu.sync_copy(outv, out_hbm.at[pl.ds(wid * BPW * N_CLASSES,
                                           BPW * N_CLASSES)])


_edge_logits = functools.partial(
    pl.kernel,
    out_type=jax.ShapeDtypeStruct((BATCH * N_CLASSES,), jnp.float32),
    mesh=plsc.VectorSubcoreMesh(core_axis_name="c", subcore_axis_name="s"),
    compiler_params=pltpu.CompilerParams(
        needs_layout_passes=False, use_tc_tiling_on_sc=False),
    scratch_types=[
        pltpu.VMEM((NCHUNK, CHUNK), jnp.int32),
        pltpu.VMEM((NCHUNK, CHUNK), jnp.int32),
        pltpu.VMEM((BPW, H_FEAT), jnp.float32),
        pltpu.VMEM((BPW, H_FEAT), jnp.float32),
        pltpu.VMEM((N_CLASSES, H_FEAT), jnp.float32),
        pltpu.VMEM((L,), jnp.float32),
        pltpu.VMEM((BPW * N_CLASSES,), jnp.float32),
        pltpu.SemaphoreType.DMA,
    ],
)(_edge_logits_body)


def kernel(src_id, dst_id, embedding, W, b):
    src2d = src_id.astype(jnp.int32).reshape(NW * NCHUNK, CHUNK)
    dst2d = dst_id.astype(jnp.int32).reshape(NW * NCHUNK, CHUNK)
    b_pad = jnp.zeros((L,), jnp.float32).at[:N_CLASSES].set(b)
    out_flat = _edge_logits(src2d, dst2d, embedding, W, b_pad)
    return out_flat.reshape(BATCH, N_CLASSES)


# speedup vs baseline: 1.6018x; 1.6018x over previous
import functools

import jax
import jax.numpy as jnp
from jax import lax
from jax.experimental import pallas as pl
from jax.experimental.pallas import tpu as pltpu
from jax.experimental.pallas import tpu_sc as plsc

NC = 2
NS = 16
L = 16
NW = NC * NS

BATCH = 16384
H_FEAT = 64
N_CLASSES = 2
BPW = BATCH // NW
G = 16
NG = BPW // G


def _edge_logits_body(src_hbm, dst_hbm, emb_hbm, w_hbm, b_hbm, out_hbm,
                      sidx, didx, sblk, dblk, wv, bv, outv, sem):
    wid = lax.axis_index("s") * NC + lax.axis_index("c")
    base = wid * BPW
    pltpu.sync_copy(src_hbm.at[pl.ds(base, BPW)], sidx)
    pltpu.sync_copy(dst_hbm.at[pl.ds(base, BPW)], didx)
    pltpu.sync_copy(w_hbm, wv)
    pltpu.sync_copy(b_hbm, bv)

    bvec = bv[...]
    nck = H_FEAT // L
    w0c = [wv[0, pl.ds(k * L, L)] for k in range(nck)]
    w1c = [wv[1, pl.ds(k * L, L)] for k in range(nck)]
    b0vec = jnp.full((L,), bvec[0], jnp.float32)
    b1vec = jnp.full((L,), bvec[1], jnp.float32)
    lane = lax.iota(jnp.int32, L)
    mask_hi = lane == (L - 1)

    def fire(g, slot):
        ivs = sidx[pl.ds(g * G, G)]
        ivd = didx[pl.ds(g * G, G)]
        for ln in range(G):
            sb = pl.multiple_of((ivs[ln] // 8) * 8, 8)
            db = pl.multiple_of((ivd[ln] // 8) * 8, 8)
            pltpu.async_copy(emb_hbm.at[pl.ds(sb, 8)], sblk.at[slot, ln], sem)
            pltpu.async_copy(emb_hbm.at[pl.ds(db, 8)], dblk.at[slot, ln], sem)
        return ivs, ivd

    def drain(slot):
        for ln in range(G):
            pltpu.make_async_copy(emb_hbm.at[pl.ds(0, 8)],
                                  sblk.at[slot, ln], sem).wait()
            pltpu.make_async_copy(emb_hbm.at[pl.ds(0, 8)],
                                  dblk.at[slot, ln], sem).wait()

    iv0 = fire(0, 0)

    def group_body(g, carry):
        ivs, ivd = carry
        slot = g % 2
        nxt = fire((g + 1) % NG, (g + 1) % 2)
        drain(slot)
        for ln in range(G):
            ssub = ivs[ln] % 8
            dsub = ivd[ln] % 8
            t0 = None
            t1 = None
            for k in range(nck):
                cd = (sblk[slot, ln, ssub, pl.ds(k * L, L)]
                      * dblk[slot, ln, dsub, pl.ds(k * L, L)])
                p0 = cd * w0c[k]
                p1 = cd * w1c[k]
                t0 = p0 if t0 is None else t0 + p0
                t1 = p1 if t1 is None else t1 + p1
            cs0 = plsc.cumsum(t0) + b0vec
            cs1 = plsc.cumsum(t1) + b1vec
            oidx = jnp.full((L,), N_CLASSES * (g * G + ln), jnp.int32)
            plsc.store_scatter(outv, [oidx], cs0, mask=mask_hi)
            plsc.store_scatter(outv, [oidx + 1], cs1, mask=mask_hi)
        return nxt

    lax.fori_loop(0, NG, group_body, iv0)
    drain(NG % 2)
    pltpu.sync_copy(outv, out_hbm.at[pl.ds(wid * BPW * N_CLASSES,
                                           BPW * N_CLASSES)])


_edge_logits = functools.partial(
    pl.kernel,
    out_type=jax.ShapeDtypeStruct((BATCH * N_CLASSES,), jnp.float32),
    mesh=plsc.VectorSubcoreMesh(core_axis_name="c", subcore_axis_name="s"),
    compiler_params=pltpu.CompilerParams(
        needs_layout_passes=False, use_tc_tiling_on_sc=True),
    scratch_types=[
        pltpu.VMEM((BPW,), jnp.int32),
        pltpu.VMEM((BPW,), jnp.int32),
        pltpu.VMEM((2, G, 8, H_FEAT), jnp.float32),
        pltpu.VMEM((2, G, 8, H_FEAT), jnp.float32),
        pltpu.VMEM((N_CLASSES, H_FEAT), jnp.float32),
        pltpu.VMEM((L,), jnp.float32),
        pltpu.VMEM((BPW * N_CLASSES,), jnp.float32),
        pltpu.SemaphoreType.DMA,
    ],
)(_edge_logits_body)


def kernel(src_id, dst_id, embedding, W, b):
    b_pad = jnp.zeros((L,), jnp.float32).at[:N_CLASSES].set(b)
    out_flat = _edge_logits(src_id.astype(jnp.int32),
                            dst_id.astype(jnp.int32), embedding, W, b_pad)
    return out_flat.reshape(BATCH, N_CLASSES)
